# baseline (speedup 1.0000x reference)
import functools

import jax
import jax.numpy as jnp
from jax import lax
from jax.experimental import pallas as pl
from jax.experimental.pallas import tpu as pltpu
from jax.experimental.pallas import tpu_sc as plsc

QUERY_MAX = 128
DOC_MAX = 128
EMBED_DIM = 128

_INFO = plsc.get_sparse_core_info()
_NC = _INFO.num_cores
_NS = _INFO.num_subcores
_L = _INFO.num_lanes
_NW = _NC * _NS

_NBUF = 6
_DEPTH = 3


@functools.lru_cache(maxsize=None)
def _make_embedder(batch: int, dim: int, lens):
    rows_per_w = batch // _NW
    assert batch % _NW == 0 and rows_per_w >= _NBUF
    max_pad = max(p for _, p in lens)
    min_real = min(r for r, _ in lens)
    mesh = plsc.VectorSubcoreMesh(core_axis_name="c", subcore_axis_name="s")

    out_type = []
    for _, pad_len in lens:
        out_type.append(
            jax.ShapeDtypeStruct((batch * pad_len, dim), jnp.float32))
    for _, pad_len in lens:
        out_type.append(jax.ShapeDtypeStruct((batch, pad_len), jnp.float32))

    @functools.partial(
        pl.kernel,
        mesh=mesh,
        out_type=out_type,
        scratch_types=[
            pltpu.VMEM((rows_per_w, max_pad), jnp.int32),
            pltpu.VMEM((rows_per_w, max_pad), jnp.float32),
            pltpu.VMEM((_NBUF, max_pad, dim), jnp.float32),
        ] + [pltpu.SemaphoreType.DMA] * (2 * _NBUF + 1),
    )
    def embed_kernel(table_hbm, *args):
        ntens = len(lens)
        tok_hbms = args[:ntens]
        out_hbms = args[ntens:2 * ntens]
        mask_hbms = args[2 * ntens:3 * ntens]
        idx_v, mask_v, rows_v = args[3 * ntens:3 * ntens + 3]
        allsems = args[3 * ntens + 3:]
        gsems = allsems[:_NBUF]
        wsems = allsems[_NBUF:2 * _NBUF]
        msem = allsems[2 * _NBUF]

        wid = lax.axis_index("s") * _NC + lax.axis_index("c")
        rbase = wid * rows_per_w

        pltpu.sync_copy(table_hbm.at[pl.ds(0, 1)],
                        rows_v.at[0, pl.ds(max_pad - 1, 1)])

        for t, (real_len, pad_len) in enumerate(lens):
            tok_hbm, out_hbm, mask_hbm = tok_hbms[t], out_hbms[t], mask_hbms[t]

            pltpu.sync_copy(tok_hbm.at[pl.ds(rbase, rows_per_w)],
                            idx_v.at[:, pl.ds(0, pad_len)])

            for b in range(_DEPTH):
                pltpu.async_copy(
                    table_hbm.at[idx_v.at[b, pl.ds(0, real_len)]],
                    rows_v.at[b, pl.ds(0, real_len)], gsems[b])

            if t == 0:
                def prefill(r, carry):
                    for b in range(_NBUF):
                        for i in range(dim // _L):
                            rows_v[b, r, pl.ds(i * _L, _L)] = (
                                rows_v[0, max_pad - 1, pl.ds(i * _L, _L)])
                    return carry

                lax.fori_loop(min_real, max_pad - 1, prefill, 0)
                for b in range(1, _NBUF):
                    for i in range(dim // _L):
                        rows_v[b, max_pad - 1, pl.ds(i * _L, _L)] = (
                            rows_v[0, max_pad - 1, pl.ds(i * _L, _L)])

            def mstep(r, carry):
                for i in range(pad_len // _L):
                    v = idx_v[r, pl.ds(i * _L, _L)]
                    mask_v[r, pl.ds(i * _L, _L)] = jnp.where(
                        v > 1, jnp.float32(1.0), jnp.float32(0.0))
                return carry

            lax.fori_loop(0, rows_per_w, mstep, 0)
            pltpu.async_copy(mask_v.at[:, pl.ds(0, pad_len)],
                             mask_hbm.at[pl.ds(rbase, rows_per_w)],
                             msem).wait()

            n_blocks = -(-rows_per_w // _NBUF)

            def ostep(o, carry):
                for j in range(_NBUF):
                    r = o * _NBUF + j

                    @pl.when(r < rows_per_w)
                    def _():
                        pltpu.make_async_copy(
                            table_hbm.at[idx_v.at[0, pl.ds(0, real_len)]],
                            rows_v.at[j, pl.ds(0, real_len)], gsems[j]).wait()
                        pltpu.async_copy(
                            rows_v.at[j],
                            out_hbm.at[pl.ds((rbase + r) * pad_len, pad_len)],
                            wsems[j])
                        nr = r + _DEPTH
                        bb = (j + _DEPTH) % _NBUF

                        @pl.when(nr < rows_per_w)
                        def _():
                            @pl.when(nr >= _NBUF)
                            def _():
                                pltpu.make_async_copy(
                                    rows_v.at[bb],
                                    out_hbm.at[pl.ds(0, pad_len)],
                                    wsems[bb]).wait()

                            pltpu.async_copy(
                                table_hbm.at[idx_v.at[nr, pl.ds(0, real_len)]],
                                rows_v.at[bb, pl.ds(0, real_len)], gsems[bb])
                return carry

            lax.fori_loop(0, n_blocks, ostep, 0)

            for b in range(_NBUF):
                pltpu.make_async_copy(
                    rows_v.at[b], out_hbm.at[pl.ds(0, pad_len)],
                    wsems[b]).wait()

    return embed_kernel


def kernel(query_tokens, doc_pos_tokens, doc_neg_tokens, embedding_table):
    batch = query_tokens.shape[0]
    dim = embedding_table.shape[1]

    def _pad(tokens, max_len):
        return jnp.pad(tokens, ((0, 0), (0, max_len - tokens.shape[1])),
                       constant_values=0)

    q = _pad(query_tokens, QUERY_MAX)
    dp = _pad(doc_pos_tokens, DOC_MAX)
    dn = _pad(doc_neg_tokens, DOC_MAX)

    lens = ((query_tokens.shape[1], QUERY_MAX),
            (doc_pos_tokens.shape[1], DOC_MAX),
            (doc_neg_tokens.shape[1], DOC_MAX))

    outs = _make_embedder(batch, dim, lens)(embedding_table, q, dp, dn)
    q_rows, dp_rows, dn_rows, q_mask, dp_mask, dn_mask = outs

    return (q_rows.reshape(batch, QUERY_MAX, dim),
            dp_rows.reshape(batch, DOC_MAX, dim),
            dn_rows.reshape(batch, DOC_MAX, dim),
            q_mask, dp_mask, dn_mask)


# speedup vs baseline: 1.0015x; 1.0015x over previous
import functools

import jax
import jax.numpy as jnp
from jax import lax
from jax.experimental import pallas as pl
from jax.experimental.pallas import tpu as pltpu
from jax.experimental.pallas import tpu_sc as plsc

QUERY_MAX = 128
DOC_MAX = 128
EMBED_DIM = 128

try:
    _INFO = plsc.get_sparse_core_info()
    _NC = _INFO.num_cores
    _NS = _INFO.num_subcores
    _L = _INFO.num_lanes
except Exception:
    _NC, _NS, _L = 2, 16, 16
_NW = _NC * _NS

_NBUF = 6
_DEPTH = 3


@functools.lru_cache(maxsize=None)
def _make_embedder(batch: int, dim: int, lens):
    rows_per_w = batch // _NW
    assert batch % _NW == 0 and rows_per_w >= _NBUF
    max_pad = max(p for _, p in lens)
    min_real = min(r for r, _ in lens)
    mesh = plsc.VectorSubcoreMesh(core_axis_name="c", subcore_axis_name="s")

    out_type = []
    for _, pad_len in lens:
        out_type.append(
            jax.ShapeDtypeStruct((batch * pad_len, dim), jnp.float32))
    for _, pad_len in lens:
        out_type.append(jax.ShapeDtypeStruct((batch, pad_len), jnp.float32))

    @functools.partial(
        pl.kernel,
        mesh=mesh,
        out_type=out_type,
        scratch_types=[
            pltpu.VMEM((rows_per_w, max_pad), jnp.int32),
            pltpu.VMEM((rows_per_w, max_pad), jnp.float32),
            pltpu.VMEM((_NBUF, max_pad, dim), jnp.float32),
        ] + [pltpu.SemaphoreType.DMA] * (2 * _NBUF + 1),
    )
    def embed_kernel(table_hbm, *args):
        ntens = len(lens)
        tok_hbms = args[:ntens]
        out_hbms = args[ntens:2 * ntens]
        mask_hbms = args[2 * ntens:3 * ntens]
        idx_v, mask_v, rows_v = args[3 * ntens:3 * ntens + 3]
        allsems = args[3 * ntens + 3:]
        gsems = allsems[:_NBUF]
        wsems = allsems[_NBUF:2 * _NBUF]
        msem = allsems[2 * _NBUF]

        wid = lax.axis_index("s") * _NC + lax.axis_index("c")
        rbase = wid * rows_per_w

        pltpu.sync_copy(table_hbm.at[pl.ds(0, 1)],
                        rows_v.at[0, pl.ds(max_pad - 1, 1)])

        for t, (real_len, pad_len) in enumerate(lens):
            tok_hbm, out_hbm, mask_hbm = tok_hbms[t], out_hbms[t], mask_hbms[t]

            pltpu.sync_copy(tok_hbm.at[pl.ds(rbase, rows_per_w)],
                            idx_v.at[:, pl.ds(0, pad_len)])

            for b in range(_DEPTH):
                pltpu.async_copy(
                    table_hbm.at[idx_v.at[b, pl.ds(0, real_len)]],
                    rows_v.at[b, pl.ds(0, real_len)], gsems[b])

            if t == 0:
                def prefill(r, carry):
                    for b in range(_NBUF):
                        for i in range(dim // _L):
                            rows_v[b, r, pl.ds(i * _L, _L)] = (
                                rows_v[0, max_pad - 1, pl.ds(i * _L, _L)])
                    return carry

                lax.fori_loop(min_real, max_pad - 1, prefill, 0)
                for b in range(1, _NBUF):
                    for i in range(dim // _L):
                        rows_v[b, max_pad - 1, pl.ds(i * _L, _L)] = (
                            rows_v[0, max_pad - 1, pl.ds(i * _L, _L)])

            def mstep(r, carry):
                for i in range(pad_len // _L):
                    v = idx_v[r, pl.ds(i * _L, _L)]
                    mask_v[r, pl.ds(i * _L, _L)] = jnp.where(
                        v > 1, jnp.float32(1.0), jnp.float32(0.0))
                return carry

            lax.fori_loop(0, rows_per_w, mstep, 0)
            pltpu.async_copy(mask_v.at[:, pl.ds(0, pad_len)],
                             mask_hbm.at[pl.ds(rbase, rows_per_w)],
                             msem).wait()

            n_blocks = -(-rows_per_w // _NBUF)

            def ostep(o, carry):
                for j in range(_NBUF):
                    r = o * _NBUF + j

                    @pl.when(r < rows_per_w)
                    def _():
                        pltpu.make_async_copy(
                            table_hbm.at[idx_v.at[0, pl.ds(0, real_len)]],
                            rows_v.at[j, pl.ds(0, real_len)], gsems[j]).wait()
                        pltpu.async_copy(
                            rows_v.at[j],
                            out_hbm.at[pl.ds((rbase + r) * pad_len, pad_len)],
                            wsems[j])
                        nr = r + _DEPTH
                        bb = (j + _DEPTH) % _NBUF

                        @pl.when(nr < rows_per_w)
                        def _():
                            @pl.when(nr >= _NBUF)
                            def _():
                                pltpu.make_async_copy(
                                    rows_v.at[bb],
                                    out_hbm.at[pl.ds(0, pad_len)],
                                    wsems[bb]).wait()

                            pltpu.async_copy(
                                table_hbm.at[idx_v.at[nr, pl.ds(0, real_len)]],
                                rows_v.at[bb, pl.ds(0, real_len)], gsems[bb])
                return carry

            lax.fori_loop(0, n_blocks, ostep, 0)

            for b in range(_NBUF):
                pltpu.make_async_copy(
                    rows_v.at[b], out_hbm.at[pl.ds(0, pad_len)],
                    wsems[b]).wait()

    return embed_kernel


def kernel(query_tokens, doc_pos_tokens, doc_neg_tokens, embedding_table):
    batch = query_tokens.shape[0]
    dim = embedding_table.shape[1]

    def _pad(tokens, max_len):
        return jnp.pad(tokens, ((0, 0), (0, max_len - tokens.shape[1])),
                       constant_values=0)

    q = _pad(query_tokens, QUERY_MAX)
    dp = _pad(doc_pos_tokens, DOC_MAX)
    dn = _pad(doc_neg_tokens, DOC_MAX)

    lens = ((query_tokens.shape[1], QUERY_MAX),
            (doc_pos_tokens.shape[1], DOC_MAX),
            (doc_neg_tokens.shape[1], DOC_MAX))

    outs = _make_embedder(batch, dim, lens)(embedding_table, q, dp, dn)
    q_rows, dp_rows, dn_rows, q_mask, dp_mask, dn_mask = outs

    return (q_rows.reshape(batch, QUERY_MAX, dim),
            dp_rows.reshape(batch, DOC_MAX, dim),
            dn_rows.reshape(batch, DOC_MAX, dim),
            q_mask, dp_mask, dn_mask)
